# baseline (speedup 1.0000x reference)
import functools

import jax
import jax.numpy as jnp
from jax import lax
from jax.experimental import pallas as pl
from jax.experimental.pallas import tpu as pltpu
from jax.experimental.pallas import tpu_sc as plsc

BATCH = 16384
D = 64
L = 16
NC = 2
NS = 16
NW = NC * NS
BPW = BATCH // NW
CHUNK = 128
NCH = BPW // CHUNK


def _body(user_hbm, item_i_hbm, item_j_hbm, eu_hbm, ei_hbm,
          out_i_hbm, out_j_hbm,
          idx_u, idx_i, idx_j, rows_u, rows_i, rows_j,
          out_i_v, out_j_v, sem):
    wid = lax.axis_index("s") * NC + lax.axis_index("c")
    base = wid * BPW

    for c in range(NCH):
        pltpu.sync_copy(user_hbm.at[pl.ds(base + c * CHUNK, CHUNK)], idx_u.at[c])
        pltpu.sync_copy(item_i_hbm.at[pl.ds(base + c * CHUNK, CHUNK)], idx_i.at[c])
        pltpu.sync_copy(item_j_hbm.at[pl.ds(base + c * CHUNK, CHUNK)], idx_j.at[c])

    copies = []
    for c in range(NCH):
        dst = pl.ds(c * CHUNK, CHUNK)
        copies.append(pltpu.async_copy(eu_hbm.at[idx_u.at[c]], rows_u.at[dst], sem))
        copies.append(pltpu.async_copy(ei_hbm.at[idx_i.at[c]], rows_i.at[dst], sem))
        copies.append(pltpu.async_copy(ei_hbm.at[idx_j.at[c]], rows_j.at[dst], sem))
    for cp in copies:
        cp.wait()

    lane = lax.iota(jnp.int32, L)
    zeros = jnp.zeros((L,), jnp.float32)

    def group(g, _):
        row0 = g * L
        t_i = zeros
        t_j = zeros
        for k in range(L):
            r = row0 + k
            pi = zeros
            pj = zeros
            for c in range(D // L):
                sl = pl.ds(c * L, L)
                u = rows_u[r, sl]
                pi = pi + u * rows_i[r, sl]
                pj = pj + u * rows_j[r, sl]
            si = jnp.sum(pi)
            sj = jnp.sum(pj)
            sel = lane == k
            t_i = jnp.where(sel, si, t_i)
            t_j = jnp.where(sel, sj, t_j)
        out_i_v[pl.ds(row0, L)] = t_i
        out_j_v[pl.ds(row0, L)] = t_j
        return 0

    lax.fori_loop(0, BPW // L, group, 0)

    pltpu.sync_copy(out_i_v, out_i_hbm.at[pl.ds(base, BPW)])
    pltpu.sync_copy(out_j_v, out_j_hbm.at[pl.ds(base, BPW)])


@jax.jit
def kernel(user, item_i, item_j, embed_user, embed_item):
    mesh = plsc.VectorSubcoreMesh(core_axis_name="c", subcore_axis_name="s")
    f32 = jnp.float32
    run = functools.partial(
        pl.kernel,
        mesh=mesh,
        compiler_params=pltpu.CompilerParams(needs_layout_passes=False, use_tc_tiling_on_sc=False),
        out_type=(
            jax.ShapeDtypeStruct((BATCH,), f32),
            jax.ShapeDtypeStruct((BATCH,), f32),
        ),
        scratch_types=[
            pltpu.VMEM((NCH, CHUNK), jnp.int32),
            pltpu.VMEM((NCH, CHUNK), jnp.int32),
            pltpu.VMEM((NCH, CHUNK), jnp.int32),
            pltpu.VMEM((BPW, D), f32),
            pltpu.VMEM((BPW, D), f32),
            pltpu.VMEM((BPW, D), f32),
            pltpu.VMEM((BPW,), f32),
            pltpu.VMEM((BPW,), f32),
            pltpu.SemaphoreType.DMA,
        ],
    )(_body)
    return run(user, item_i, item_j, embed_user, embed_item)
